# baseline (speedup 1.0000x reference)
import functools

import jax
import jax.numpy as jnp
from jax import lax
from jax.experimental import pallas as pl
from jax.experimental.pallas import tpu as pltpu
from jax.experimental.pallas import tpu_sc as plsc

VOCAB = 100000
VPAD = 100096
D = 64
C = 4
B = 4096
S = 200

NC, NS, L = 2, 16, 16
NW = NC * NS
NPAIR = C // 2
NBLK = NW // NPAIR
ROWS_PER_W = B // NBLK
UNROLL = 8
RS0 = 96
RS1 = S - RS0

VB = 12544


def _fold_body(et_ref, w_ref, b_ref, out_ref):
    t = lax.dot_general(
        w_ref[...], et_ref[...], (((1,), (0,)), ((), ())),
        preferred_element_type=jnp.float32,
    )
    rows = []
    for c in range(C):
        rows.append((t[c : c + 1, :] + b_ref[c]) * (1.0 / S))
    bb = [
        (u + jnp.uint32(0x7FFF) + ((u >> jnp.uint32(16)) & jnp.uint32(1)))
        >> jnp.uint32(16)
        for u in (lax.bitcast_convert_type(r, jnp.uint32) for r in rows)
    ]
    packed = jnp.concatenate(
        [bb[0] | (bb[1] << jnp.uint32(16)), bb[2] | (bb[3] << jnp.uint32(16))],
        axis=0,
    )
    out_ref[...] = lax.bitcast_convert_type(packed, jnp.int32)


def _fold_table(embedding_t, fc_w, fc_b):
    return pl.pallas_call(
        _fold_body,
        grid=(pl.cdiv(VOCAB, VB),),
        in_specs=[
            pl.BlockSpec((D, VB), lambda i: (0, i)),
            pl.BlockSpec((C, D), lambda i: (0, 0)),
            pl.BlockSpec(memory_space=pltpu.SMEM),
        ],
        out_specs=pl.BlockSpec((NPAIR, VB), lambda i: (0, i)),
        out_shape=jax.ShapeDtypeStruct((NPAIR, VPAD), jnp.int32),
    )(embedding_t, fc_w, fc_b)


_sc_mesh = plsc.VectorSubcoreMesh(core_axis_name="c", subcore_axis_name="s")


@functools.partial(
    pl.kernel,
    out_type=jax.ShapeDtypeStruct((C, B), jnp.float32),
    mesh=_sc_mesh,
    compiler_params=pltpu.CompilerParams(needs_layout_passes=False),
    scratch_types=[
        pltpu.VMEM((VPAD,), jnp.int32),
        pltpu.VMEM((RS0, 128), jnp.int32),
        pltpu.VMEM((RS1, 128), jnp.int32),
        pltpu.VMEM((2 * ROWS_PER_W,), jnp.float32),
        pltpu.SemaphoreType.DMA,
        pltpu.SemaphoreType.DMA,
        pltpu.SemaphoreType.DMA,
        pltpu.SemaphoreType.DMA,
    ],
)
def _pool_kernel(
    tbl_hbm, idst_hbm, out_hbm, col_v, idxa_v, idxb_v, out_v,
    col_sem, ia_sem, ib_sem, out_sem,
):
    wid = lax.axis_index("s") * NC + lax.axis_index("c")
    pair = wid // NBLK
    blk = wid % NBLK
    NCH = 34
    CH = VPAD // NCH
    rot = wid % NCH
    col_cps = []
    for k in range(NCH):
        off = ((rot + k) % NCH) * CH
        col_cps.append(
            pltpu.async_copy(
                tbl_hbm.at[pair].at[pl.ds(off, CH)], col_v.at[pl.ds(off, CH)], col_sem
            )
        )
    b0 = blk * ROWS_PER_W
    cp_a = pltpu.async_copy(idst_hbm.at[pl.ds(0, RS0), pl.ds(b0, 128)], idxa_v, ia_sem)
    cp_b = pltpu.async_copy(idst_hbm.at[pl.ds(RS0, RS1), pl.ds(b0, 128)], idxb_v, ib_sem)
    for cp in col_cps:
        cp.wait()

    def quarter(idx_ref, nrows, g, a0, a1):
        def s_body(i, accs, _ref=idx_ref):
            q0, q1 = accs
            s0 = i * UNROLL
            for j in range(UNROLL):
                idx = _ref[s0 + j, pl.ds(pl.multiple_of(g * L, L), L)]
                packed = plsc.load_gather(col_v, [idx])
                q0 = q0 + plsc.bitcast(packed << jnp.int32(16), jnp.float32)
                q1 = q1 + plsc.bitcast(packed & jnp.int32(-65536), jnp.float32)
            return q0, q1

        return lax.fori_loop(0, nrows // UNROLL, s_body, (a0, a1))

    out_cps = []
    for slab in range(ROWS_PER_W // 128):
        cp_a.wait()
        cp_b.wait()

        def phase_a(g, carry, _slab=slab):
            zero = jnp.zeros((L,), jnp.float32)
            a0, a1 = quarter(idxa_v, RS0, g, zero, zero)
            row = pl.multiple_of(_slab * 128 + g * L, L)
            out_v[pl.ds(row, L)] = a0
            out_v[pl.ds(ROWS_PER_W + row, L)] = a1
            return carry

        lax.fori_loop(0, 128 // L, phase_a, 0)
        if slab == 0:
            cp_a = pltpu.async_copy(
                idst_hbm.at[pl.ds(0, RS0), pl.ds(b0 + 128, 128)], idxa_v, ia_sem
            )

        def phase_b(g, carry, _slab=slab):
            row = pl.multiple_of(_slab * 128 + g * L, L)
            a0 = out_v[pl.ds(row, L)]
            a1 = out_v[pl.ds(ROWS_PER_W + row, L)]
            a0, a1 = quarter(idxb_v, RS1, g, a0, a1)
            out_v[pl.ds(row, L)] = a0
            out_v[pl.ds(ROWS_PER_W + row, L)] = a1
            return carry

        lax.fori_loop(0, 128 // L, phase_b, 0)
        if slab == 0:
            cp_b = pltpu.async_copy(
                idst_hbm.at[pl.ds(RS0, RS1), pl.ds(b0 + 128, 128)], idxb_v, ib_sem
            )
        out_cps.append(
            pltpu.async_copy(
                out_v.at[pl.ds(slab * 128, 128)],
                out_hbm.at[2 * pair, pl.ds(b0 + slab * 128, 128)],
                out_sem,
            )
        )
        out_cps.append(
            pltpu.async_copy(
                out_v.at[pl.ds(ROWS_PER_W + slab * 128, 128)],
                out_hbm.at[2 * pair + 1, pl.ds(b0 + slab * 128, 128)],
                out_sem,
            )
        )
    for cp in out_cps:
        cp.wait()


def kernel(input_ids, embedding, fc_w, fc_b):
    tbl = _fold_table(embedding.T, fc_w, fc_b)
    idst = input_ids.astype(jnp.int32).T
    out_t = _pool_kernel(tbl, idst)
    return out_t.T


# speedup vs baseline: 1.0543x; 1.0543x over previous
import functools

import jax
import jax.numpy as jnp
from jax import lax
from jax.experimental import pallas as pl
from jax.experimental.pallas import tpu as pltpu
from jax.experimental.pallas import tpu_sc as plsc

VOCAB = 100000
VPAD = 100096
D = 64
C = 4
B = 4096
S = 200

NC, NS, L = 2, 16, 16
NW = NC * NS
NPAIR = C // 2
NBLK = NW // NPAIR
ROWS_PER_W = B // NBLK
UNROLL = 8
RS0 = 96
RS1 = S - RS0

VB = 25088


def _fold_body(et_ref, w_ref, b_ref, out_ref):
    t = lax.dot_general(
        w_ref[...], et_ref[...], (((1,), (0,)), ((), ())),
        preferred_element_type=jnp.float32,
    )
    rows = []
    for c in range(C):
        rows.append((t[c : c + 1, :] + b_ref[c]) * (1.0 / S))
    bb = [
        (u + jnp.uint32(0x7FFF) + ((u >> jnp.uint32(16)) & jnp.uint32(1)))
        >> jnp.uint32(16)
        for u in (lax.bitcast_convert_type(r, jnp.uint32) for r in rows)
    ]
    packed = jnp.concatenate(
        [bb[0] | (bb[1] << jnp.uint32(16)), bb[2] | (bb[3] << jnp.uint32(16))],
        axis=0,
    )
    out_ref[...] = lax.bitcast_convert_type(packed, jnp.int32)


def _fold_table(embedding_t, fc_w, fc_b):
    return pl.pallas_call(
        _fold_body,
        grid=(pl.cdiv(VOCAB, VB),),
        in_specs=[
            pl.BlockSpec((D, VB), lambda i: (0, i)),
            pl.BlockSpec((C, D), lambda i: (0, 0)),
            pl.BlockSpec(memory_space=pltpu.SMEM),
        ],
        out_specs=pl.BlockSpec((NPAIR, VB), lambda i: (0, i)),
        out_shape=jax.ShapeDtypeStruct((NPAIR, VPAD), jnp.int32),
    )(embedding_t, fc_w, fc_b)


_sc_mesh = plsc.VectorSubcoreMesh(core_axis_name="c", subcore_axis_name="s")


@functools.partial(
    pl.kernel,
    out_type=jax.ShapeDtypeStruct((C, B), jnp.float32),
    mesh=_sc_mesh,
    compiler_params=pltpu.CompilerParams(needs_layout_passes=False),
    scratch_types=[
        pltpu.VMEM((VPAD,), jnp.int32),
        pltpu.VMEM((RS0, 128), jnp.int32),
        pltpu.VMEM((RS1, 128), jnp.int32),
        pltpu.VMEM((2 * ROWS_PER_W,), jnp.float32),
        pltpu.SemaphoreType.DMA,
        pltpu.SemaphoreType.DMA,
        pltpu.SemaphoreType.DMA,
        pltpu.SemaphoreType.DMA,
    ],
)
def _pool_kernel(
    tbl_hbm, idst_hbm, out_hbm, col_v, idxa_v, idxb_v, out_v,
    col_sem, ia_sem, ib_sem, out_sem,
):
    wid = lax.axis_index("s") * NC + lax.axis_index("c")
    pair = wid // NBLK
    blk = wid % NBLK
    NCH = 17
    CH = VPAD // NCH
    rot = wid % NCH
    col_cps = []
    for k in range(NCH):
        off = ((rot + k) % NCH) * CH
        col_cps.append(
            pltpu.async_copy(
                tbl_hbm.at[pair].at[pl.ds(off, CH)], col_v.at[pl.ds(off, CH)], col_sem
            )
        )
    b0 = blk * ROWS_PER_W
    cp_a = pltpu.async_copy(idst_hbm.at[pl.ds(0, RS0), pl.ds(b0, 128)], idxa_v, ia_sem)
    cp_b = pltpu.async_copy(idst_hbm.at[pl.ds(RS0, RS1), pl.ds(b0, 128)], idxb_v, ib_sem)
    for cp in col_cps:
        cp.wait()

    def quarter(idx_ref, nrows, g, a0, a1):
        def s_body(i, accs, _ref=idx_ref):
            q0, q1 = accs
            s0 = i * UNROLL
            for j in range(UNROLL):
                idx = _ref[s0 + j, pl.ds(pl.multiple_of(g * L, L), L)]
                packed = plsc.load_gather(col_v, [idx])
                q0 = q0 + plsc.bitcast(packed << jnp.int32(16), jnp.float32)
                q1 = q1 + plsc.bitcast(packed & jnp.int32(-65536), jnp.float32)
            return q0, q1

        return lax.fori_loop(0, nrows // UNROLL, s_body, (a0, a1))

    out_cps = []
    for slab in range(ROWS_PER_W // 128):
        cp_a.wait()
        cp_b.wait()

        def phase_a(g, carry, _slab=slab):
            zero = jnp.zeros((L,), jnp.float32)
            a0, a1 = quarter(idxa_v, RS0, g, zero, zero)
            row = pl.multiple_of(_slab * 128 + g * L, L)
            out_v[pl.ds(row, L)] = a0
            out_v[pl.ds(ROWS_PER_W + row, L)] = a1
            return carry

        lax.fori_loop(0, 128 // L, phase_a, 0)
        if slab == 0:
            cp_a = pltpu.async_copy(
                idst_hbm.at[pl.ds(0, RS0), pl.ds(b0 + 128, 128)], idxa_v, ia_sem
            )

        def phase_b(g, carry, _slab=slab):
            row = pl.multiple_of(_slab * 128 + g * L, L)
            a0 = out_v[pl.ds(row, L)]
            a1 = out_v[pl.ds(ROWS_PER_W + row, L)]
            a0, a1 = quarter(idxb_v, RS1, g, a0, a1)
            out_v[pl.ds(row, L)] = a0
            out_v[pl.ds(ROWS_PER_W + row, L)] = a1
            return carry

        lax.fori_loop(0, 128 // L, phase_b, 0)
        if slab == 0:
            cp_b = pltpu.async_copy(
                idst_hbm.at[pl.ds(RS0, RS1), pl.ds(b0 + 128, 128)], idxb_v, ib_sem
            )
        out_cps.append(
            pltpu.async_copy(
                out_v.at[pl.ds(slab * 128, 128)],
                out_hbm.at[2 * pair, pl.ds(b0 + slab * 128, 128)],
                out_sem,
            )
        )
        out_cps.append(
            pltpu.async_copy(
                out_v.at[pl.ds(ROWS_PER_W + slab * 128, 128)],
                out_hbm.at[2 * pair + 1, pl.ds(b0 + slab * 128, 128)],
                out_sem,
            )
        )
    for cp in out_cps:
        cp.wait()


def kernel(input_ids, embedding, fc_w, fc_b):
    tbl = _fold_table(embedding.T, fc_w, fc_b)
    idst = input_ids.astype(jnp.int32).T
    out_t = _pool_kernel(tbl, idst)
    return out_t.T
